# Initial kernel scaffold; baseline (speedup 1.0000x reference)
import jax
import jax.numpy as jnp
from jax.experimental import pallas as pl


def kernel(x, edge_index, batch, W1l, b1, W1r, W2l, b2, W2r):
    raise NotImplementedError("write your pallas kernel here")


# speedup vs baseline: 3.4072x; 3.4072x over previous
import functools

import jax
import jax.numpy as jnp
from jax import lax
from jax.experimental import pallas as pl
from jax.experimental.pallas import tpu as pltpu
from jax.experimental.pallas import tpu_sc as plsc

N = 10000
E = 320000
F = 128
H = 128
B = 128

NC = 2
NS = 16
NW = NC * NS
CHUNK = 64

N_PAD = 10176
W = F + 16
NBUF = 2
T_EDGE = 158
E_PAD = NW * CHUNK * T_EDGE

ROWS_PER_TILE = N_PAD // NS


def _sc_aggregate(table, src, dst, zeros):
    mesh = plsc.VectorSubcoreMesh(core_axis_name="c", subcore_axis_name="s")

    @functools.partial(
        pl.kernel,
        mesh=mesh,
        out_type=jax.ShapeDtypeStruct((NC, N_PAD, W), jnp.float32),
        scratch_types=[
            pltpu.VMEM_SHARED((N_PAD, W), jnp.float32),
            pltpu.VMEM((T_EDGE + NBUF, CHUNK), jnp.int32),
            pltpu.VMEM((T_EDGE, CHUNK), jnp.int32),
            [pltpu.VMEM((CHUNK, W), jnp.float32) for _ in range(NBUF)],
            [pltpu.SemaphoreType.DMA for _ in range(NBUF)],
        ],
        compiler_params=pltpu.CompilerParams(use_tc_tiling_on_sc=False),
    )
    def k(table_hbm, src_hbm, dst_hbm, zeros_hbm, out_hbm,
          acc_sh, src_v, dst_v, rows_v, sems):
        cid = lax.axis_index("c")
        sid = lax.axis_index("s")
        wid = sid * NC + cid

        pltpu.sync_copy(src_hbm.at[wid], src_v)
        pltpu.sync_copy(dst_hbm.at[wid], dst_v)

        pltpu.sync_copy(zeros_hbm.at[pl.ds(sid * ROWS_PER_TILE, ROWS_PER_TILE)],
                        acc_sh.at[pl.ds(sid * ROWS_PER_TILE, ROWS_PER_TILE)])

        for b in range(NBUF):
            pltpu.async_copy(table_hbm.at[src_v.at[b]], rows_v[b], sems[b])

        plsc.subcore_barrier()

        def body(g, _):
            for b in range(NBUF):
                t = g * NBUF + b
                pltpu.make_async_copy(table_hbm.at[src_v.at[t]],
                                      rows_v[b], sems[b]).wait()
                pltpu.sync_copy(rows_v[b], acc_sh.at[dst_v.at[t]], add=True)
                pltpu.async_copy(table_hbm.at[src_v.at[t + NBUF]],
                                 rows_v[b], sems[b])
            return ()

        lax.fori_loop(0, T_EDGE // NBUF, body, ())

        for b in range(NBUF):
            pltpu.make_async_copy(table_hbm.at[src_v.at[0]],
                                  rows_v[b], sems[b]).wait()
        plsc.subcore_barrier()

        pltpu.sync_copy(acc_sh.at[pl.ds(sid * ROWS_PER_TILE, ROWS_PER_TILE)],
                        out_hbm.at[cid, pl.ds(sid * ROWS_PER_TILE,
                                              ROWS_PER_TILE)])

    return k(table, src, dst, zeros)


BLK = 1272
GRID = N_PAD // BLK


def _tc_layer1(agg_parts, x, W1l, b1, W1r):
    def body(agg_ref, x_ref, wl_ref, b_ref, wr_ref, out_ref):
        a = agg_ref[0] + agg_ref[1]
        deg = jnp.clip(a[:, F:F + 1], 1.0, None)
        agg = a[:, :F] / deg
        h = jnp.dot(agg, wl_ref[...], preferred_element_type=jnp.float32)
        h = h + b_ref[...]
        h = h + jnp.dot(x_ref[...], wr_ref[...],
                        preferred_element_type=jnp.float32)
        h = jnp.maximum(h, 0.0)
        out_ref[:, :F] = h
        out_ref[:, F:] = (jax.lax.broadcasted_iota(
            jnp.int32, (BLK, W - F), 1) == 0).astype(jnp.float32)

    return pl.pallas_call(
        body,
        grid=(GRID,),
        in_specs=[
            pl.BlockSpec((NC, BLK, W), lambda i: (0, i, 0)),
            pl.BlockSpec((BLK, F), lambda i: (i, 0)),
            pl.BlockSpec((F, H), lambda i: (0, 0)),
            pl.BlockSpec((1, H), lambda i: (0, 0)),
            pl.BlockSpec((F, H), lambda i: (0, 0)),
        ],
        out_specs=pl.BlockSpec((BLK, W), lambda i: (i, 0)),
        out_shape=jax.ShapeDtypeStruct((N_PAD, W), jnp.float32),
    )(agg_parts, x, W1l, b1, W1r)


def _tc_layer2_pool(agg_parts, h1_aug, batch2d, W2l, b2, W2r):
    def body(agg_ref, h1_ref, batch_ref, wl_ref, b_ref, wr_ref, out_ref,
             sums_acc, cnts_acc):
        i = pl.program_id(0)

        a = agg_ref[0] + agg_ref[1]
        deg = jnp.clip(a[:, F:F + 1], 1.0, None)
        agg = a[:, :F] / deg
        h = jnp.dot(agg, wl_ref[...], preferred_element_type=jnp.float32)
        h = h + b_ref[...]
        h = h + jnp.dot(h1_ref[:, :F], wr_ref[...],
                        preferred_element_type=jnp.float32)

        onehot_t = (batch_ref[0] == jax.lax.broadcasted_iota(
            jnp.int32, (B, BLK), 0)).astype(jnp.float32)

        psum = jnp.dot(onehot_t, h, preferred_element_type=jnp.float32)
        pcnt = jnp.sum(onehot_t, axis=1, keepdims=True)

        @pl.when(i == 0)
        def _():
            sums_acc[...] = jnp.zeros_like(sums_acc)
            cnts_acc[...] = jnp.zeros_like(cnts_acc)

        sums_acc[...] += psum
        cnts_acc[...] += pcnt


        @pl.when(i == GRID - 1)
        def _():
            out_ref[...] = sums_acc[...] / jnp.clip(cnts_acc[...], 1.0, None)

    return pl.pallas_call(
        body,
        grid=(GRID,),
        in_specs=[
            pl.BlockSpec((NC, BLK, W), lambda i: (0, i, 0)),
            pl.BlockSpec((BLK, W), lambda i: (i, 0)),
            pl.BlockSpec((1, 1, BLK), lambda i: (i, 0, 0)),
            pl.BlockSpec((H, H), lambda i: (0, 0)),
            pl.BlockSpec((1, H), lambda i: (0, 0)),
            pl.BlockSpec((H, H), lambda i: (0, 0)),
        ],
        out_specs=pl.BlockSpec((B, H), lambda i: (0, 0)),
        out_shape=jax.ShapeDtypeStruct((B, H), jnp.float32),
        scratch_shapes=[
            pltpu.VMEM((B, H), jnp.float32),
            pltpu.VMEM((B, 1), jnp.float32),
        ],
    )(agg_parts, h1_aug, batch2d, W2l, b2, W2r)


def kernel(x, edge_index, batch, W1l, b1, W1r, W2l, b2, W2r):
    src = edge_index[0]
    dst = edge_index[1]
    pad_e = E_PAD - E
    src_p = jnp.concatenate([src, jnp.full((pad_e,), N, jnp.int32)])
    dst_p = jnp.concatenate([dst, jnp.full((pad_e,), N, jnp.int32)])
    src_p = src_p.reshape(NW, T_EDGE, CHUNK)
    src_p = jnp.concatenate(
        [src_p, jnp.full((NW, NBUF, CHUNK), N, jnp.int32)], axis=1)
    dst_p = dst_p.reshape(NW, T_EDGE, CHUNK)

    x_aug = jnp.zeros((N_PAD, W), jnp.float32)
    x_aug = x_aug.at[:N, :F].set(x)
    x_aug = x_aug.at[:N, F].set(1.0)

    x_pad = jnp.zeros((N_PAD, F), jnp.float32).at[:N].set(x)

    batch_p = jnp.full((N_PAD,), B, jnp.int32).at[:N].set(batch)
    batch_p = batch_p.reshape(GRID, 1, BLK)

    zeros = jnp.zeros((N_PAD, W), jnp.float32)
    b1r = b1.reshape(1, H)
    b2r = b2.reshape(1, H)

    agg1 = _sc_aggregate(x_aug, src_p, dst_p, zeros)
    h1_aug = _tc_layer1(agg1, x_pad, W1l, b1r, W1r)

    agg2 = _sc_aggregate(h1_aug, src_p, dst_p, zeros)
    return _tc_layer2_pool(agg2, h1_aug, batch_p, W2l, b2r, W2r)
